# Initial kernel scaffold; baseline (speedup 1.0000x reference)
import jax
import jax.numpy as jnp
from jax.experimental import pallas as pl


def kernel(x, edge_index, batch, W, b):
    raise NotImplementedError("write your pallas kernel here")


# speedup vs baseline: 4.8903x; 4.8903x over previous
import jax
import jax.numpy as jnp
from jax import lax
from jax.experimental import pallas as pl
from jax.experimental.pallas import tpu as pltpu

_SEGS = 64
_N = 50000
_D = 1024
_BLOCK = 1000
_NBLK = _N // _BLOCK


def _fused_body(x_ref, batch_ref, w_ref, b_ref, out_ref, cnt_ref):
    i = pl.program_id(0)

    @pl.when(i == 0)
    def _():
        out_ref[...] = jnp.zeros_like(out_ref)
        cnt_ref[...] = jnp.zeros_like(cnt_ref)

    xb = x_ref[...]
    h = lax.dot_general(xb, w_ref[...], (((1,), (1,)), ((), ())),
                        preferred_element_type=jnp.float32)
    bidx = batch_ref[0]
    seg = lax.broadcasted_iota(jnp.int32, (_SEGS, _BLOCK), 0)
    onehot = (bidx == seg).astype(jnp.float32)
    psum = lax.dot_general(onehot, h, (((1,), (0,)), ((), ())),
                           preferred_element_type=jnp.float32)
    pcnt = jnp.sum(onehot, axis=1, keepdims=True)
    acc = out_ref[...] + psum
    cnt = cnt_ref[...] + pcnt
    out_ref[...] = acc
    cnt_ref[...] = cnt

    @pl.when(i == _NBLK - 1)
    def _():
        out_ref[...] = (acc + cnt * b_ref[...]) / jnp.maximum(cnt, 1.0)


def kernel(x, edge_index, batch, W, b):
    batch3 = batch.reshape(_NBLK, 1, _BLOCK)
    b2 = b.reshape(1, 2)
    out = pl.pallas_call(
        _fused_body,
        grid=(_NBLK,),
        in_specs=[
            pl.BlockSpec((_BLOCK, _D), lambda i: (i, 0)),
            pl.BlockSpec((1, 1, _BLOCK), lambda i: (i, 0, 0)),
            pl.BlockSpec((2, _D), lambda i: (0, 0)),
            pl.BlockSpec((1, 2), lambda i: (0, 0)),
        ],
        out_specs=pl.BlockSpec((_SEGS, 2), lambda i: (0, 0)),
        out_shape=jax.ShapeDtypeStruct((_SEGS, 2), jnp.float32),
        scratch_shapes=[pltpu.VMEM((_SEGS, 1), jnp.float32)],
        compiler_params=pltpu.CompilerParams(
            dimension_semantics=("arbitrary",)),
    )(x, batch3, W, b2)
    return out


# speedup vs baseline: 6.0209x; 1.2312x over previous
import jax
import jax.numpy as jnp
from jax import lax
from jax.experimental import pallas as pl
from jax.experimental.pallas import tpu as pltpu

_SEGS = 64
_N = 50000
_D = 1024
_BLOCK = 2000
_NBLK = _N // _BLOCK


def _fused_body(x_ref, batch_ref, w_ref, b_ref, out_ref, cnt_ref):
    i = pl.program_id(0)

    @pl.when(i == 0)
    def _():
        out_ref[...] = jnp.zeros_like(out_ref)
        cnt_ref[...] = jnp.zeros_like(cnt_ref)

    xb = x_ref[...]
    h = lax.dot_general(xb, w_ref[...], (((1,), (1,)), ((), ())),
                        preferred_element_type=jnp.float32)
    bidx = batch_ref[0]
    seg = lax.broadcasted_iota(jnp.int32, (_SEGS, _BLOCK), 0)
    onehot = (bidx == seg).astype(jnp.float32)
    psum = lax.dot_general(onehot, h, (((1,), (0,)), ((), ())),
                           preferred_element_type=jnp.float32)
    pcnt = jnp.sum(onehot, axis=1, keepdims=True)
    acc = out_ref[...] + psum
    cnt = cnt_ref[...] + pcnt
    out_ref[...] = acc
    cnt_ref[...] = cnt

    @pl.when(i == _NBLK - 1)
    def _():
        out_ref[...] = (acc + cnt * b_ref[...]) / jnp.maximum(cnt, 1.0)


def kernel(x, edge_index, batch, W, b):
    batch3 = batch.reshape(_NBLK, 1, _BLOCK)
    b2 = b.reshape(1, 2)
    out = pl.pallas_call(
        _fused_body,
        grid=(_NBLK,),
        in_specs=[
            pl.BlockSpec((_BLOCK, _D), lambda i: (i, 0)),
            pl.BlockSpec((1, 1, _BLOCK), lambda i: (i, 0, 0)),
            pl.BlockSpec((2, _D), lambda i: (0, 0)),
            pl.BlockSpec((1, 2), lambda i: (0, 0)),
        ],
        out_specs=pl.BlockSpec((_SEGS, 2), lambda i: (0, 0)),
        out_shape=jax.ShapeDtypeStruct((_SEGS, 2), jnp.float32),
        scratch_shapes=[pltpu.VMEM((_SEGS, 1), jnp.float32)],
        compiler_params=pltpu.CompilerParams(
            dimension_semantics=("arbitrary",)),
    )(x, batch3, W, b2)
    return out
